# baseline (speedup 1.0000x reference)
import jax
import jax.numpy as jnp
from jax.experimental import pallas as pl
from jax.experimental.pallas import tpu as pltpu

_N = 5000
_NPAD = 5120
_C = 256
_T = _NPAD // _C
_BIG = 1e9


def _oversegment_kernel(box1_ref, b2c_ref, cov1_ref, cov2_ref, cols_ref, rows_ref):
    b1x1 = box1_ref[0:1, 0:1]
    b1y1 = box1_ref[0:1, 1:2]
    b1x2 = box1_ref[0:1, 2:3]
    b1y2 = box1_ref[0:1, 3:4]

    x1c = b2c_ref[0:1, :]
    y1c = b2c_ref[1:2, :]
    x2c = b2c_ref[2:3, :]
    y2c = b2c_ref[3:4, :]

    ix1 = jnp.maximum(b1x1, x1c)
    iy1 = jnp.maximum(b1y1, y1c)
    ix2 = jnp.minimum(b1x2, x2c)
    iy2 = jnp.minimum(b1y2, y2c)
    ia = jnp.maximum(ix2 - ix1 + 1.0, 0.0) * jnp.maximum(iy2 - iy1 + 1.0, 0.0)
    valid = ia > 0.0
    cols_ref[0:1, :] = jnp.where(valid, ix1, _BIG)
    cols_ref[1:2, :] = jnp.where(valid, iy1, _BIG)
    cols_ref[2:3, :] = jnp.where(valid, ix2, -_BIG) + 1.0
    cols_ref[3:4, :] = jnp.where(valid, iy2, -_BIG) + 1.0

    b2area = (x2c - x1c + 1.0) * (y2c - y1c + 1.0)
    cov2_ref[...] = ia / b2area

    sum_ia = jnp.sum(ia)

    zvec = jnp.zeros((1, _C), jnp.float32)

    def outer(i, carry):
        a_up, a_diag = carry
        rs = pl.ds(i * _C, _C)
        rows_ref[0:_C, :] = jnp.broadcast_to(cols_ref[0:1, rs], (_C, _C)).T
        rows_ref[_C : 2 * _C, :] = jnp.broadcast_to(cols_ref[1:2, rs], (_C, _C)).T
        rows_ref[2 * _C : 3 * _C, :] = jnp.broadcast_to(cols_ref[2:3, rs], (_C, _C)).T
        rows_ref[3 * _C : 4 * _C, :] = jnp.broadcast_to(cols_ref[3:4, rs], (_C, _C)).T

        def inner(j, carry2):
            su, sd = carry2
            cs = pl.ds(j * _C, _C)
            cx1 = cols_ref[0:1, cs]
            cy1 = cols_ref[1:2, cs]
            cx2p = cols_ref[2:3, cs]
            cy2p = cols_ref[3:4, cs]
            t = jnp.zeros((1, _C), jnp.float32)
            for s in range(0, _C, 32):
                x_ext = jnp.maximum(rows_ref[2 * _C + s : 2 * _C + s + 32, :], cx2p) - jnp.maximum(
                    rows_ref[s : s + 32, :], cx1
                )
                y_ext = jnp.maximum(rows_ref[3 * _C + s : 3 * _C + s + 32, :], cy2p) - jnp.maximum(
                    rows_ref[_C + s : _C + s + 32, :], cy1
                )
                p = jnp.maximum(x_ext, 0.0) * jnp.maximum(y_ext, 0.0)
                t = t + jnp.sum(p, axis=0, keepdims=True)
            is_diag = j == i
            su = su + jnp.where(is_diag, zvec, t)
            sd = sd + jnp.where(is_diag, t, zvec)
            return su, sd

        return jax.lax.fori_loop(i, _T, inner, (a_up, a_diag))

    a_up, a_diag = jax.lax.fori_loop(0, _T, outer, (zvec, zvec))
    s_up = jnp.sum(a_up)
    s_diag = jnp.sum(a_diag)

    tri = s_up + (s_diag - sum_ia) * 0.5
    b1area = (b1x2 - b1x1 + 1.0) * (b1y2 - b1y1 + 1.0)
    cov1_ref[...] = (sum_ia - tri) / b1area


def kernel(box1, box2):
    pad = jnp.tile(
        jnp.array([[_BIG, _BIG, -_BIG, -_BIG]], dtype=jnp.float32),
        (_NPAD - _N, 1),
    )
    b2p = jnp.concatenate([box2.astype(jnp.float32), pad], axis=0)
    b2c = b2p.T

    cov1, cov2 = pl.pallas_call(
        _oversegment_kernel,
        out_shape=[
            jax.ShapeDtypeStruct((1, 1), jnp.float32),
            jax.ShapeDtypeStruct((1, _NPAD), jnp.float32),
        ],
        scratch_shapes=[
            pltpu.VMEM((8, _NPAD), jnp.float32),
            pltpu.VMEM((4 * _C, _C), jnp.float32),
        ],
    )(box1.astype(jnp.float32), b2c)

    return cov1.reshape(1), cov2[0, :_N]
